# baseline (speedup 1.0000x reference)
import functools

import jax
import jax.numpy as jnp
from jax import lax
from jax.experimental import pallas as pl
from jax.experimental.pallas import tpu as pltpu
from jax.experimental.pallas import tpu_sc as plsc

B = 16384
NUM = 256
F = 26
V = 100000
D = 64
H = 1024

R = B * F
NC = 2
NS = 16
NW = NC * NS
SL = 128
SPW = R // (NW * SL)
SLICES_PER_CHUNK = 4
CHUNK_ROWS = SLICES_PER_CHUNK * SL
CHUNKS = SPW // SLICES_PER_CHUNK


def _sc_gather(tbl, idx2d):
    mesh = plsc.VectorSubcoreMesh(core_axis_name="c", subcore_axis_name="s")

    @functools.partial(
        pl.kernel,
        mesh=mesh,
        compiler_params=pltpu.CompilerParams(use_tc_tiling_on_sc=False),
        out_type=jax.ShapeDtypeStruct((R, D), jnp.float32),
        scratch_types=[
            pltpu.VMEM((SLICES_PER_CHUNK, SL), jnp.int32),
            pltpu.VMEM((CHUNK_ROWS, D), jnp.float32),
            pltpu.SemaphoreType.DMA,
        ],
    )
    def k(tbl_hbm, idx_hbm, out_hbm, idxc_v, rows_v, sem):
        wid = lax.axis_index("s") * NC + lax.axis_index("c")
        slice_base = wid * SPW
        row_base = slice_base * SL

        def chunk_body(c, carry):
            s0 = slice_base + c * SLICES_PER_CHUNK
            pltpu.sync_copy(idx_hbm.at[pl.ds(s0, SLICES_PER_CHUNK)], idxc_v)
            copies = [
                pltpu.async_copy(
                    tbl_hbm.at[idxc_v.at[j]],
                    rows_v.at[pl.ds(j * SL, SL)],
                    sem,
                )
                for j in range(SLICES_PER_CHUNK)
            ]
            for cp in copies:
                cp.wait()
            pltpu.sync_copy(
                rows_v, out_hbm.at[pl.ds(row_base + c * CHUNK_ROWS, CHUNK_ROWS)]
            )
            return carry

        lax.fori_loop(0, CHUNKS, chunk_body, 0)

    return k(tbl, idx2d)


def _mlp_body(xn_ref, emb_ref, w1a_ref, w1b_ref, b1_ref, w2t_ref, b2_ref, out_ref):
    h = jnp.dot(xn_ref[...], w1a_ref[...], preferred_element_type=jnp.float32)
    h = h + jnp.dot(emb_ref[...], w1b_ref[...], preferred_element_type=jnp.float32)
    h = jnp.maximum(h + b1_ref[...], 0.0)
    s = jnp.sum(h * w2t_ref[...], axis=1, keepdims=True) + b2_ref[0]
    out_ref[...] = jax.nn.sigmoid(s)


def _tc_mlp(x_num, emb, w1a, w1b, b1r, w2r, b2r, bb=512):
    grid = (B // bb,)
    return pl.pallas_call(
        _mlp_body,
        grid=grid,
        in_specs=[
            pl.BlockSpec((bb, NUM), lambda i: (i, 0)),
            pl.BlockSpec((bb, F * D), lambda i: (i, 0)),
            pl.BlockSpec((NUM, H), lambda i: (0, 0)),
            pl.BlockSpec((F * D, H), lambda i: (0, 0)),
            pl.BlockSpec((1, H), lambda i: (0, 0)),
            pl.BlockSpec((1, H), lambda i: (0, 0)),
            pl.BlockSpec(memory_space=pltpu.SMEM),
        ],
        out_specs=pl.BlockSpec((bb, 1), lambda i: (i, 0)),
        out_shape=jax.ShapeDtypeStruct((B, 1), jnp.float32),
    )(x_num, emb, w1a, w1b, b1r, w2r, b2r)


def kernel(x_num, x_cat, tables, W1, b1, W2, b2):
    idx = x_cat.astype(jnp.int32) + (jnp.arange(F, dtype=jnp.int32) * V)[None, :]
    idx2d = idx.reshape(R // SL, SL)
    tbl = tables.reshape(F * V, D)
    rows = _sc_gather(tbl, idx2d)
    emb = rows.reshape(B, F * D)
    out = _tc_mlp(
        x_num,
        emb,
        W1[:NUM],
        W1[NUM:],
        b1.reshape(1, H),
        W2.reshape(1, H),
        b2.reshape(1),
    )
    return out


# speedup vs baseline: 1.6924x; 1.6924x over previous
import functools

import jax
import jax.numpy as jnp
from jax import lax
from jax.experimental import pallas as pl
from jax.experimental.pallas import tpu as pltpu
from jax.experimental.pallas import tpu_sc as plsc

B = 16384
NUM = 256
F = 26
V = 100000
D = 64
H = 1024

VB = 8192
NJ = (V + VB - 1) // VB
PF = NJ * (VB // 2)
RP = F * PF

R = B * F
NC = 2
NS = 16
NW = NC * NS
SL = 128
SPW = R // (NW * SL)
SPC = 4
CR = SPC * SL
CHUNKS = SPW // SPC


def _repack_body(tin_ref, out_ref):
    x = tin_ref[0]
    eye = (
        lax.broadcasted_iota(jnp.int32, (D, D), 0)
        == lax.broadcasted_iota(jnp.int32, (D, D), 1)
    ).astype(jnp.float32)
    y = lax.dot_general(
        x, eye, (((0,), (0,)), ((), ())), preferred_element_type=jnp.float32
    )
    out_ref[0, :, 0:D] = y[0 : VB // 2]
    out_ref[0, :, D : 2 * D] = y[VB // 2 : VB]


def _tc_repack(tbl_t):
    return pl.pallas_call(
        _repack_body,
        grid=(F, NJ),
        in_specs=[pl.BlockSpec((1, D, VB), lambda f, j: (f, 0, j))],
        out_specs=pl.BlockSpec((1, VB // 2, 2 * D), lambda f, j: (f, j, 0)),
        out_shape=jax.ShapeDtypeStruct((F, PF, 2 * D), jnp.float32),
    )(tbl_t)


def _sc_gather(tbl2d, idx2d):
    mesh = plsc.VectorSubcoreMesh(core_axis_name="c", subcore_axis_name="s")

    @functools.partial(
        pl.kernel,
        mesh=mesh,
        compiler_params=pltpu.CompilerParams(use_tc_tiling_on_sc=True),
        out_type=jax.ShapeDtypeStruct((R, 2 * D), jnp.float32),
        scratch_types=[
            pltpu.VMEM((SPC, SL), jnp.int32),
            pltpu.VMEM((CR, 2 * D), jnp.float32),
            pltpu.SemaphoreType.DMA,
        ],
    )
    def k(tbl_hbm, idx_hbm, out_hbm, idxc_v, rows_v, sem):
        wid = lax.axis_index("s") * NC + lax.axis_index("c")
        slice_base = wid * SPW
        row_base = slice_base * SL

        def chunk_body(c, carry):
            s0 = slice_base + c * SPC
            pltpu.sync_copy(idx_hbm.at[pl.ds(s0, SPC)], idxc_v)
            copies = [
                pltpu.async_copy(
                    tbl_hbm.at[idxc_v.at[j]],
                    rows_v.at[pl.ds(j * SL, SL)],
                    sem,
                )
                for j in range(SPC)
            ]
            for cp in copies:
                cp.wait()
            pltpu.sync_copy(
                rows_v, out_hbm.at[pl.ds(row_base + c * CR, CR)]
            )
            return carry

        lax.fori_loop(0, CHUNKS, chunk_body, 0)

    return k(tbl2d, idx2d)


def _mlp_body(
    xn_ref, emb_ref, half_ref, w1a_ref, w1b_ref, b1_ref, w2t_ref, b2_ref, out_ref
):
    h = jnp.dot(xn_ref[...], w1a_ref[...], preferred_element_type=jnp.float32)
    pieces = []
    for f in range(F):
        slab = emb_ref[f]
        hf = half_ref[f][:, None]
        pieces.append(jnp.where(hf == 1, slab[:, D : 2 * D], slab[:, 0:D]))
    e = jnp.concatenate(pieces, axis=1)
    h = h + jnp.dot(e, w1b_ref[...], preferred_element_type=jnp.float32)
    h = jnp.maximum(h + b1_ref[...], 0.0)
    s = jnp.sum(h * w2t_ref[...], axis=1, keepdims=True) + b2_ref[0]
    out_ref[...] = jax.nn.sigmoid(s)


def _tc_mlp(x_num, emb3, half, w1a, w1b, b1r, w2r, b2r, bb=512):
    return pl.pallas_call(
        _mlp_body,
        grid=(B // bb,),
        in_specs=[
            pl.BlockSpec((bb, NUM), lambda i: (i, 0)),
            pl.BlockSpec((F, bb, 2 * D), lambda i: (0, i, 0)),
            pl.BlockSpec((F, bb), lambda i: (0, i)),
            pl.BlockSpec((NUM, H), lambda i: (0, 0)),
            pl.BlockSpec((F * D, H), lambda i: (0, 0)),
            pl.BlockSpec((1, H), lambda i: (0, 0)),
            pl.BlockSpec((1, H), lambda i: (0, 0)),
            pl.BlockSpec(memory_space=pltpu.SMEM),
        ],
        out_specs=pl.BlockSpec((bb, 1), lambda i: (i, 0)),
        out_shape=jax.ShapeDtypeStruct((B, 1), jnp.float32),
    )(x_num, emb3, half, w1a, w1b, b1r, w2r, b2r)


def kernel(x_num, x_cat, tables, W1, b1, W2, b2):
    tbl_t = jnp.transpose(tables, (0, 2, 1))
    xc = jnp.transpose(x_cat.astype(jnp.int32), (1, 0))
    fcol = lax.broadcasted_iota(jnp.int32, (F, B), 0)
    rows = fcol * PF + (xc >> 13) * (VB // 2) + (xc & (VB // 2 - 1))
    half = (xc >> 12) & 1
    tbl2d = _tc_repack(tbl_t).reshape(RP, 2 * D)
    g = _sc_gather(tbl2d, rows.reshape(R // SL, SL))
    emb3 = g.reshape(F, B, 2 * D)
    out = _tc_mlp(
        x_num,
        emb3,
        half,
        W1[:NUM],
        W1[NUM:],
        b1.reshape(1, H),
        W2.reshape(1, H),
        b2.reshape(1),
    )
    return out
